# baseline (speedup 1.0000x reference)
import jax
import jax.numpy as jnp
from jax.experimental import pallas as pl
from jax.experimental.pallas import tpu as pltpu

NV = 32
CH = 8
BR = 4096


def _onehot_body(x_ref, o_ref):
    idx = x_ref[:, CH - 1:CH].astype(jnp.int32)
    iota = jax.lax.broadcasted_iota(jnp.int32, (BR, NV), 1)
    o_ref[:, :] = (iota == idx).astype(jnp.float32)


def kernel(network):
    B, T, C = network.shape
    N = B * T
    x = network.reshape(N, C)
    grid = N // BR
    out = pl.pallas_call(
        _onehot_body,
        grid=(grid,),
        in_specs=[pl.BlockSpec((BR, C), lambda i: (i, 0))],
        out_specs=pl.BlockSpec((BR, NV), lambda i: (i, 0)),
        out_shape=jax.ShapeDtypeStruct((N, NV), jnp.float32),
    )(x)
    return out.reshape(B, T, NV)
